# Initial kernel scaffold; baseline (speedup 1.0000x reference)
import jax
import jax.numpy as jnp
from jax.experimental import pallas as pl


def kernel(z, embedding):
    raise NotImplementedError("write your pallas kernel here")


# speedup vs baseline: 1.2800x; 1.2800x over previous
import jax
import jax.numpy as jnp
from jax import lax
from jax.experimental import pallas as pl
from jax.experimental.pallas import tpu as pltpu

_BETA = 0.25
_EPS = 1e-10


def _vq_body(e_ref, z_ref, z2_ref, e2_ref, zq_ref, idx_ref, loss_ref, pp_ref,
             counts_ref, lacc_ref):
    i = pl.program_id(0)
    nsteps = pl.num_programs(0)

    @pl.when(i == 0)
    def _init():
        counts_ref[...] = jnp.zeros_like(counts_ref)
        lacc_ref[0, 0] = 0.0

    emb = e_ref[...]
    zblk = z_ref[0]
    z2 = z2_ref[0]
    e2 = e2_ref[...]
    kcodes, tb = e2.shape[0], z2.shape[1]
    cdim = zblk.shape[0]
    ntok = nsteps * tb

    mm = lax.dot_general(emb, zblk, (((1,), (0,)), ((), ())),
                         preferred_element_type=jnp.float32,
                         precision=lax.Precision.DEFAULT)
    dist = (z2 + e2) - 2.0 * mm
    minv = jnp.min(dist, axis=0, keepdims=True)
    iota = lax.broadcasted_iota(jnp.int32, (kcodes, tb), 0)
    idx = jnp.min(jnp.where(dist == minv, iota, kcodes),
                  axis=0, keepdims=True)
    idx_ref[0] = idx

    onehot = jnp.where(iota == idx, 1.0, 0.0)
    counts_ref[...] += jnp.sum(onehot, axis=1, keepdims=True)
    zq = lax.dot_general(emb, onehot, (((0,), (0,)), ((), ())),
                         preferred_element_type=jnp.float32,
                         precision=lax.Precision.DEFAULT)
    zq_ref[0] = zq
    diff = zblk - zq
    lacc_ref[0, 0] += jnp.sum(diff * diff)

    @pl.when(i == nsteps - 1)
    def _fin():
        avg = counts_ref[...] * (1.0 / ntok)
        ent = jnp.sum(avg * jnp.log(avg + _EPS))
        pp_ref[0, 0] = jnp.exp(-ent)
        loss_ref[0, 0] = lacc_ref[0, 0] * (_BETA / (ntok * cdim))


def kernel(z, embedding):
    B, C, D, H, W = z.shape
    K = embedding.shape[0]
    S = D * H * W
    N = B * S
    TB = 512
    NB = N // TB
    SB = S // TB

    z3 = z.reshape(B, C, S)
    z_flat = jnp.transpose(z, (0, 2, 3, 4, 1)).reshape(-1, C)
    z2 = jnp.sum(z_flat ** 2, axis=1, keepdims=True).reshape(NB, 1, TB)
    e2 = jnp.sum(embedding ** 2, axis=1, keepdims=True)

    out_shape = (
        jax.ShapeDtypeStruct((B, C, S), jnp.float32),
        jax.ShapeDtypeStruct((NB, 1, TB), jnp.int32),
        jax.ShapeDtypeStruct((1, 1), jnp.float32),
        jax.ShapeDtypeStruct((1, 1), jnp.float32),
    )
    in_specs = [
        pl.BlockSpec((K, C), lambda i: (0, 0)),
        pl.BlockSpec((1, C, TB), lambda i: (i // SB, 0, i % SB)),
        pl.BlockSpec((1, 1, TB), lambda i: (i, 0, 0)),
        pl.BlockSpec((K, 1), lambda i: (0, 0)),
    ]
    out_specs = (
        pl.BlockSpec((1, C, TB), lambda i: (i // SB, 0, i % SB)),
        pl.BlockSpec((1, 1, TB), lambda i: (i, 0, 0)),
        pl.BlockSpec(memory_space=pltpu.SMEM),
        pl.BlockSpec(memory_space=pltpu.SMEM),
    )
    zq3, idxb, loss, pp = pl.pallas_call(
        _vq_body,
        grid=(NB,),
        in_specs=in_specs,
        out_specs=out_specs,
        out_shape=out_shape,
        scratch_shapes=[pltpu.VMEM((K, 1), jnp.float32),
                        pltpu.SMEM((1, 1), jnp.float32)],
    )(embedding, z3, z2, e2)

    z_q = zq3.reshape(B, C, D, H, W)
    indices = idxb.reshape(B, D, H, W)
    return (z_q, indices, loss[0, 0], pp[0, 0])
